# Initial kernel scaffold; baseline (speedup 1.0000x reference)
import jax
import jax.numpy as jnp
from jax.experimental import pallas as pl


def kernel(pair, Wq, bq, Wk, bk, Wv, bv, Wg, bg, Wo, bo, gamma, beta):
    raise NotImplementedError("write your pallas kernel here")


# speedup vs baseline: 1.2887x; 1.2887x over previous
import jax
import jax.numpy as jnp
from jax.experimental import pallas as pl

N = 256
C = 128
H = 4
Ch = C // H
IB = 8


def _tri_kernel(x_ref, wall_ref, ball_ref, wo_ref, bo_ref, gb_ref, o_ref):
    x = x_ref[0].reshape(IB * N, C)
    gamma = gb_ref[0:1, :]
    beta = gb_ref[1:2, :]

    mu = jnp.mean(x, axis=1, keepdims=True)
    xc = x - mu
    var = jnp.mean(xc * xc, axis=1, keepdims=True)
    xn = xc * jax.lax.rsqrt(var + 1e-5) * gamma + beta

    xnb = xn.astype(jnp.bfloat16)
    qkvg = (
        jnp.dot(xnb, wall_ref[...], preferred_element_type=jnp.float32)
        + ball_ref[...]
    )

    outs = []
    for ii in range(IB):
        row = qkvg[ii * N : (ii + 1) * N]
        q = row[:, 0:C]
        k = row[:, C : 2 * C]
        v = row[:, 2 * C : 3 * C]
        g = row[:, 3 * C : 4 * C]
        qb = q.astype(jnp.bfloat16)
        kb = k.astype(jnp.bfloat16)
        vb = v.astype(jnp.bfloat16)
        ohs = []
        for h in range(H):
            qh = qb[:, h * Ch : (h + 1) * Ch]
            kh = kb[:, h * Ch : (h + 1) * Ch]
            vh = vb[:, h * Ch : (h + 1) * Ch]
            logits = jax.lax.dot_general(
                qh, kh, (((1,), (1,)), ((), ())),
                preferred_element_type=jnp.float32,
            )
            m = jnp.max(logits, axis=1, keepdims=True)
            p = jnp.exp(logits - m)
            s = jnp.sum(p, axis=1, keepdims=True)
            oh = (
                jnp.dot(p.astype(jnp.bfloat16), vh,
                        preferred_element_type=jnp.float32)
                / s
            )
            ohs.append(oh)
        o = jnp.concatenate(ohs, axis=1)
        gate = jax.nn.sigmoid(g)
        outs.append(o * gate)

    of = jnp.concatenate(outs, axis=0)
    out = (
        jnp.dot(of.astype(jnp.bfloat16), wo_ref[...],
                preferred_element_type=jnp.float32)
        + bo_ref[...]
        + x
    )
    o_ref[0] = out.reshape(IB, N, C)


def kernel(pair, Wq, bq, Wk, bk, Wv, bv, Wg, bg, Wo, bo, gamma, beta):
    scale = Ch ** -0.5
    wall = jnp.concatenate(
        [Wq * scale, Wk, Wv, Wg], axis=1
    ).astype(jnp.bfloat16)
    ball = jnp.concatenate([bq * scale, bk, bv, bg])[None, :]
    wo = Wo.astype(jnp.bfloat16)
    bo2 = bo[None, :]
    gb = jnp.stack([gamma, beta])

    out = pl.pallas_call(
        _tri_kernel,
        grid=(N // IB,),
        in_specs=[
            pl.BlockSpec((1, IB, N, C), lambda ib: (0, ib, 0, 0)),
            pl.BlockSpec((C, 4 * C), lambda ib: (0, 0)),
            pl.BlockSpec((1, 4 * C), lambda ib: (0, 0)),
            pl.BlockSpec((C, C), lambda ib: (0, 0)),
            pl.BlockSpec((1, C), lambda ib: (0, 0)),
            pl.BlockSpec((2, C), lambda ib: (0, 0)),
        ],
        out_specs=pl.BlockSpec((1, IB, N, C), lambda ib: (0, ib, 0, 0)),
        out_shape=jax.ShapeDtypeStruct(pair.shape, jnp.float32),
    )(pair, wall, ball, wo, bo2, gb)
    return out


# speedup vs baseline: 4.3341x; 3.3631x over previous
import jax
import jax.numpy as jnp
from jax.experimental import pallas as pl

N = 256
C = 128
H = 4
Ch = C // H
IB = 8


def _tri_kernel(x_ref, wall_ref, ball_ref, wo_ref, bo_ref, gb_ref,
                mask_ref, o_ref):
    x = x_ref[0].reshape(IB * N, C)
    gamma = gb_ref[0:1, :]
    beta = gb_ref[1:2, :]

    mu = jnp.mean(x, axis=1, keepdims=True)
    xc = x - mu
    var = jnp.mean(xc * xc, axis=1, keepdims=True)
    xn = xc * jax.lax.rsqrt(var + 1e-5) * gamma + beta

    qkvg = (
        jnp.dot(xn.astype(jnp.bfloat16), wall_ref[...],
                preferred_element_type=jnp.float32)
        + ball_ref[...]
    ).astype(jnp.bfloat16)

    mask = mask_ref[...]
    outs = []
    for ii in range(IB):
        row = qkvg[ii * N : (ii + 1) * N]
        q = row[:, 0:C]
        k = row[:, C : 2 * C]
        v = row[:, 2 * C : 3 * C]
        g = row[:, 3 * C : 4 * C]

        k_bd = jnp.concatenate([k, k, k, k], axis=0) * mask
        logits = jax.lax.dot_general(
            q, k_bd, (((1,), (1,)), ((), ())),
            preferred_element_type=jnp.float32,
        )
        p = jnp.exp(logits.astype(jnp.bfloat16))

        v_bd = jnp.concatenate([v, v, v, v], axis=0) * mask
        w_av = jnp.concatenate([v_bd, mask], axis=1)
        o_s = jnp.dot(p, w_av, preferred_element_type=jnp.float32)
        o = o_s[:, 0:C] / o_s[:, C : 2 * C]

        gate = jax.nn.sigmoid(g)
        outs.append((o * gate).astype(jnp.bfloat16))

    of = jnp.concatenate(outs, axis=0)
    out = (
        jnp.dot(of, wo_ref[...], preferred_element_type=jnp.float32)
        + bo_ref[...]
        + x
    )
    o_ref[0] = out.reshape(IB, N, C)


def kernel(pair, Wq, bq, Wk, bk, Wv, bv, Wg, bg, Wo, bo, gamma, beta):
    scale = Ch ** -0.5
    wall = jnp.concatenate(
        [Wq * scale, Wk, Wv, Wg], axis=1
    ).astype(jnp.bfloat16)
    ball = jnp.concatenate(
        [bq * scale, bk, bv, bg]
    )[None, :].astype(jnp.bfloat16)
    wo = Wo.astype(jnp.bfloat16)
    bo2 = bo[None, :]
    gb = jnp.stack([gamma, beta])
    mask = (
        jnp.arange(H * N)[:, None] // N == jnp.arange(C)[None, :] // Ch
    ).astype(jnp.bfloat16)

    out = pl.pallas_call(
        _tri_kernel,
        grid=(N // IB,),
        in_specs=[
            pl.BlockSpec((1, IB, N, C), lambda ib: (0, ib, 0, 0)),
            pl.BlockSpec((C, 4 * C), lambda ib: (0, 0)),
            pl.BlockSpec((1, 4 * C), lambda ib: (0, 0)),
            pl.BlockSpec((C, C), lambda ib: (0, 0)),
            pl.BlockSpec((1, C), lambda ib: (0, 0)),
            pl.BlockSpec((2, C), lambda ib: (0, 0)),
            pl.BlockSpec((H * N, C), lambda ib: (0, 0)),
        ],
        out_specs=pl.BlockSpec((1, IB, N, C), lambda ib: (0, ib, 0, 0)),
        out_shape=jax.ShapeDtypeStruct(pair.shape, jnp.float32),
    )(pair, wall, ball, wo, bo2, gb, mask)
    return out


# speedup vs baseline: 4.6665x; 1.0767x over previous
import jax
import jax.numpy as jnp
from jax.experimental import pallas as pl

N = 256
C = 128
H = 4
Ch = C // H
IB = 16


def _tri_kernel(x_ref, wall_ref, ball_ref, wo_ref, bo_ref, gb_ref,
                mask_ref, o_ref):
    x = x_ref[0].reshape(IB * N, C)
    gamma = gb_ref[0:1, :]
    beta = gb_ref[1:2, :]

    mu = jnp.mean(x, axis=1, keepdims=True)
    xc = x - mu
    var = jnp.mean(xc * xc, axis=1, keepdims=True)
    xn = xc * jax.lax.rsqrt(var + 1e-5) * gamma + beta

    qkvg = (
        jnp.dot(xn.astype(jnp.bfloat16), wall_ref[...],
                preferred_element_type=jnp.float32)
        + ball_ref[...]
    ).astype(jnp.bfloat16)

    mask = mask_ref[...]
    outs = []
    for ii in range(IB):
        row = qkvg[ii * N : (ii + 1) * N]
        q = row[:, 0:C]
        k = row[:, C : 2 * C]
        v = row[:, 2 * C : 3 * C]
        g = row[:, 3 * C : 4 * C]

        k_bd = jnp.concatenate([k, k, k, k], axis=0) * mask
        logits = jax.lax.dot_general(
            q, k_bd, (((1,), (1,)), ((), ())),
            preferred_element_type=jnp.float32,
        )
        p = jnp.exp(logits.astype(jnp.bfloat16))

        v_bd = jnp.concatenate([v, v, v, v], axis=0) * mask
        w_av = jnp.concatenate([v_bd, mask], axis=1)
        o_s = jnp.dot(p, w_av, preferred_element_type=jnp.float32)
        o = o_s[:, 0:C] / o_s[:, C : 2 * C]

        gate = jax.nn.sigmoid(g)
        outs.append((o * gate).astype(jnp.bfloat16))

    of = jnp.concatenate(outs, axis=0)
    out = (
        jnp.dot(of, wo_ref[...], preferred_element_type=jnp.float32)
        + bo_ref[...]
        + x
    )
    o_ref[0] = out.reshape(IB, N, C)


def kernel(pair, Wq, bq, Wk, bk, Wv, bv, Wg, bg, Wo, bo, gamma, beta):
    scale = Ch ** -0.5
    wall = jnp.concatenate(
        [Wq * scale, Wk, Wv, Wg], axis=1
    ).astype(jnp.bfloat16)
    ball = jnp.concatenate(
        [bq * scale, bk, bv, bg]
    )[None, :].astype(jnp.bfloat16)
    wo = Wo.astype(jnp.bfloat16)
    bo2 = bo[None, :]
    gb = jnp.stack([gamma, beta])
    mask = (
        jnp.arange(H * N)[:, None] // N == jnp.arange(C)[None, :] // Ch
    ).astype(jnp.bfloat16)

    out = pl.pallas_call(
        _tri_kernel,
        grid=(N // IB,),
        in_specs=[
            pl.BlockSpec((1, IB, N, C), lambda ib: (0, ib, 0, 0)),
            pl.BlockSpec((C, 4 * C), lambda ib: (0, 0)),
            pl.BlockSpec((1, 4 * C), lambda ib: (0, 0)),
            pl.BlockSpec((C, C), lambda ib: (0, 0)),
            pl.BlockSpec((1, C), lambda ib: (0, 0)),
            pl.BlockSpec((2, C), lambda ib: (0, 0)),
            pl.BlockSpec((H * N, C), lambda ib: (0, 0)),
        ],
        out_specs=pl.BlockSpec((1, IB, N, C), lambda ib: (0, ib, 0, 0)),
        out_shape=jax.ShapeDtypeStruct(pair.shape, jnp.float32),
    )(pair, wall, ball, wo, bo2, gb, mask)
    return out


# speedup vs baseline: 5.0973x; 1.0923x over previous
import jax
import jax.numpy as jnp
from jax.experimental import pallas as pl

N = 256
C = 128
H = 4
Ch = C // H
IB = 32


def _tri_kernel(x_ref, wall_ref, ball_ref, wo_ref, bo_ref, gb_ref,
                mask_ref, o_ref):
    x = x_ref[0].reshape(IB * N, C)
    gamma = gb_ref[0:1, :]
    beta = gb_ref[1:2, :]

    mu = jnp.mean(x, axis=1, keepdims=True)
    xc = x - mu
    var = jnp.mean(xc * xc, axis=1, keepdims=True)
    xn = xc * jax.lax.rsqrt(var + 1e-5) * gamma + beta

    qkvg = (
        jnp.dot(xn.astype(jnp.bfloat16), wall_ref[...],
                preferred_element_type=jnp.float32)
        + ball_ref[...]
    ).astype(jnp.bfloat16)

    mask = mask_ref[...]
    outs = []
    for ii in range(IB):
        row = qkvg[ii * N : (ii + 1) * N]
        q = row[:, 0:C]
        k = row[:, C : 2 * C]
        v = row[:, 2 * C : 3 * C]
        g = row[:, 3 * C : 4 * C]

        k_bd = jnp.concatenate([k, k, k, k], axis=0) * mask
        logits = jax.lax.dot_general(
            q, k_bd, (((1,), (1,)), ((), ())),
            preferred_element_type=jnp.float32,
        )
        p = jnp.exp(logits.astype(jnp.bfloat16))

        v_bd = jnp.concatenate([v, v, v, v], axis=0) * mask
        w_av = jnp.concatenate([v_bd, mask], axis=1)
        o_s = jnp.dot(p, w_av, preferred_element_type=jnp.float32)
        o = o_s[:, 0:C] / o_s[:, C : 2 * C]

        gate = jax.nn.sigmoid(g)
        outs.append((o * gate).astype(jnp.bfloat16))

    of = jnp.concatenate(outs, axis=0)
    out = (
        jnp.dot(of, wo_ref[...], preferred_element_type=jnp.float32)
        + bo_ref[...]
        + x
    )
    o_ref[0] = out.reshape(IB, N, C)


def kernel(pair, Wq, bq, Wk, bk, Wv, bv, Wg, bg, Wo, bo, gamma, beta):
    scale = Ch ** -0.5
    wall = jnp.concatenate(
        [Wq * scale, Wk, Wv, Wg], axis=1
    ).astype(jnp.bfloat16)
    ball = jnp.concatenate(
        [bq * scale, bk, bv, bg]
    )[None, :].astype(jnp.bfloat16)
    wo = Wo.astype(jnp.bfloat16)
    bo2 = bo[None, :]
    gb = jnp.stack([gamma, beta])
    mask = (
        jnp.arange(H * N)[:, None] // N == jnp.arange(C)[None, :] // Ch
    ).astype(jnp.bfloat16)

    out = pl.pallas_call(
        _tri_kernel,
        grid=(N // IB,),
        in_specs=[
            pl.BlockSpec((1, IB, N, C), lambda ib: (0, ib, 0, 0)),
            pl.BlockSpec((C, 4 * C), lambda ib: (0, 0)),
            pl.BlockSpec((1, 4 * C), lambda ib: (0, 0)),
            pl.BlockSpec((C, C), lambda ib: (0, 0)),
            pl.BlockSpec((1, C), lambda ib: (0, 0)),
            pl.BlockSpec((2, C), lambda ib: (0, 0)),
            pl.BlockSpec((H * N, C), lambda ib: (0, 0)),
        ],
        out_specs=pl.BlockSpec((1, IB, N, C), lambda ib: (0, ib, 0, 0)),
        out_shape=jax.ShapeDtypeStruct(pair.shape, jnp.float32),
    )(pair, wall, ball, wo, bo2, gb, mask)
    return out


# speedup vs baseline: 5.4505x; 1.0693x over previous
import jax
import jax.numpy as jnp
from jax.experimental import pallas as pl

N = 256
C = 128
H = 4
Ch = C // H
IB = 32


def _tri_kernel(x_ref, wall_ref, ball_ref, wo_ref, bo_ref, gb_ref,
                mask_ref, o_ref):
    x = x_ref[0].reshape(IB * N, C)
    gamma = gb_ref[0:1, :]
    beta = gb_ref[1:2, :]

    mu = jnp.mean(x, axis=1, keepdims=True)
    m2 = jnp.mean(x * x, axis=1, keepdims=True)
    r = jax.lax.rsqrt(m2 - mu * mu + 1e-5)
    xn = ((x - mu) * (r * gamma) + beta).astype(jnp.bfloat16)

    qkvg = (
        jnp.dot(xn, wall_ref[...], preferred_element_type=jnp.float32)
        + ball_ref[...]
    ).astype(jnp.bfloat16)

    mask = mask_ref[...]
    outs = []
    for ii in range(IB):
        row = qkvg[ii * N : (ii + 1) * N]
        q = row[:, 0:C]
        k = row[:, C : 2 * C]
        v = row[:, 2 * C : 3 * C]
        g = row[:, 3 * C : 4 * C]

        k_bd = jnp.concatenate([k, k, k, k], axis=0) * mask
        logits = jax.lax.dot_general(
            q, k_bd, (((1,), (1,)), ((), ())),
            preferred_element_type=jnp.float32,
        )
        p = jnp.exp(logits.astype(jnp.bfloat16))

        v_bd = jnp.concatenate([v, v, v, v], axis=0) * mask
        w_av = jnp.concatenate([v_bd, mask], axis=1)
        o_s = jnp.dot(p, w_av, preferred_element_type=jnp.float32)
        o = o_s[:, 0:C] / o_s[:, C : 2 * C]

        gate = jax.nn.sigmoid(g)
        outs.append((o * gate).astype(jnp.bfloat16))

    of = jnp.concatenate(outs, axis=0)
    out = (
        jnp.dot(of, wo_ref[...], preferred_element_type=jnp.float32)
        + bo_ref[...]
        + x
    )
    o_ref[0] = out.reshape(IB, N, C)


def kernel(pair, Wq, bq, Wk, bk, Wv, bv, Wg, bg, Wo, bo, gamma, beta):
    scale = Ch ** -0.5
    wall = jnp.concatenate(
        [Wq * scale, Wk, Wv, Wg], axis=1
    ).astype(jnp.bfloat16)
    ball = jnp.concatenate(
        [bq * scale, bk, bv, bg]
    )[None, :].astype(jnp.bfloat16)
    wo = Wo.astype(jnp.bfloat16)
    bo2 = bo[None, :]
    gb = jnp.stack([gamma, beta])
    mask = (
        jnp.arange(H * N)[:, None] // N == jnp.arange(C)[None, :] // Ch
    ).astype(jnp.bfloat16)

    out = pl.pallas_call(
        _tri_kernel,
        grid=(N // IB,),
        in_specs=[
            pl.BlockSpec((1, IB, N, C), lambda ib: (0, ib, 0, 0)),
            pl.BlockSpec((C, 4 * C), lambda ib: (0, 0)),
            pl.BlockSpec((1, 4 * C), lambda ib: (0, 0)),
            pl.BlockSpec((C, C), lambda ib: (0, 0)),
            pl.BlockSpec((1, C), lambda ib: (0, 0)),
            pl.BlockSpec((2, C), lambda ib: (0, 0)),
            pl.BlockSpec((H * N, C), lambda ib: (0, 0)),
        ],
        out_specs=pl.BlockSpec((1, IB, N, C), lambda ib: (0, ib, 0, 0)),
        out_shape=jax.ShapeDtypeStruct(pair.shape, jnp.float32),
    )(pair, wall, ball, wo, bo2, gb, mask)
    return out


# speedup vs baseline: 5.9219x; 1.0865x over previous
import numpy as np

import jax
import jax.numpy as jnp
from jax.experimental import pallas as pl

N = 256
C = 128
H = 4
Ch = C // H
IB = 32

_MASK = np.repeat(
    np.eye(H, dtype=np.float32), N, axis=0
).repeat(Ch, axis=1)


def _tri_kernel(x_ref, wq_ref, wk_ref, wv_ref, wg_ref, wo_ref, vec_ref,
                mask_ref, o_ref):
    scale = Ch ** -0.5
    x = x_ref[0].reshape(IB * N, C)
    vecs = vec_ref[...]
    gamma = vecs[5:6, :]
    beta = vecs[6:7, :]

    wall = jnp.concatenate(
        [wq_ref[...] * scale, wk_ref[...], wv_ref[...], wg_ref[...]],
        axis=1,
    ).astype(jnp.bfloat16)
    ball = jnp.concatenate(
        [vecs[0:1, :] * scale, vecs[1:2, :], vecs[2:3, :], vecs[3:4, :]],
        axis=1,
    )
    wo = wo_ref[...].astype(jnp.bfloat16)
    bo = vecs[4:5, :]

    mu = jnp.mean(x, axis=1, keepdims=True)
    m2 = jnp.mean(x * x, axis=1, keepdims=True)
    r = jax.lax.rsqrt(m2 - mu * mu + 1e-5)
    xn = ((x - mu) * (r * gamma) + beta).astype(jnp.bfloat16)

    qkvg = (
        jnp.dot(xn, wall, preferred_element_type=jnp.float32) + ball
    ).astype(jnp.bfloat16)

    mask = mask_ref[...]
    outs = []
    for ii in range(IB):
        row = qkvg[ii * N : (ii + 1) * N]
        q = row[:, 0:C]
        k = row[:, C : 2 * C]
        v = row[:, 2 * C : 3 * C]
        g = row[:, 3 * C : 4 * C]

        k_bd = jnp.concatenate([k, k, k, k], axis=0) * mask
        logits = jax.lax.dot_general(
            q, k_bd, (((1,), (1,)), ((), ())),
            preferred_element_type=jnp.float32,
        )
        p = jnp.exp(logits.astype(jnp.bfloat16))

        v_bd = jnp.concatenate([v, v, v, v], axis=0) * mask
        w_av = jnp.concatenate([v_bd, mask], axis=1)
        o_s = jnp.dot(p, w_av, preferred_element_type=jnp.float32)
        o = o_s[:, 0:C] / o_s[:, C : 2 * C]

        gate = jax.nn.sigmoid(g)
        outs.append((o * gate).astype(jnp.bfloat16))

    of = jnp.concatenate(outs, axis=0)
    out = (
        jnp.dot(of, wo, preferred_element_type=jnp.float32) + bo + x
    )
    o_ref[0] = out.reshape(IB, N, C)


def kernel(pair, Wq, bq, Wk, bk, Wv, bv, Wg, bg, Wo, bo, gamma, beta):
    vecs = jnp.stack(
        [bq, bk, bv, bg, bo, gamma, beta, jnp.zeros_like(bo)]
    )
    mask = jnp.asarray(_MASK, dtype=jnp.bfloat16)

    full = lambda shape: [
        pl.BlockSpec(shape, lambda ib: tuple(0 for _ in shape))
    ]
    out = pl.pallas_call(
        _tri_kernel,
        grid=(N // IB,),
        in_specs=[
            pl.BlockSpec((1, IB, N, C), lambda ib: (0, ib, 0, 0)),
            *(full((C, C)) * 5),
            *full((8, C)),
            *full((H * N, C)),
        ],
        out_specs=pl.BlockSpec((1, IB, N, C), lambda ib: (0, ib, 0, 0)),
        out_shape=jax.ShapeDtypeStruct(pair.shape, jnp.float32),
    )(pair, Wq, Wk, Wv, Wg, Wo, vecs, mask)
    return out
